# baseline (speedup 1.0000x reference)
import functools

import jax
import jax.numpy as jnp
from jax import lax
from jax.experimental import pallas as pl
from jax.experimental.pallas import tpu as pltpu
from jax.experimental.pallas import tpu_sc as plsc

F = 26
FIELD = 100000
B = 16384
NC, NS, L = 2, 16, 16
NW = NC * NS
BPW = B // NW
E = BPW * F
OUTV = BPW // L
TOTAL = F * FIELD

_mesh = plsc.VectorSubcoreMesh(core_axis_name="c", subcore_axis_name="s")


@functools.partial(
    pl.kernel,
    out_type=jax.ShapeDtypeStruct((B,), jnp.float32),
    mesh=_mesh,
    scratch_types=[
        pltpu.VMEM((E,), jnp.int32),
        pltpu.VMEM((E,), jnp.float32),
        pltpu.VMEM((BPW,), jnp.float32),
        pltpu.SemaphoreType.DMA,
    ],
)
def _fl_kernel(xr_hbm, table_hbm, out_hbm, idx_v, vals_v, acc_v, sem):
    wid = lax.axis_index("s") * NC + lax.axis_index("c")
    base = wid * BPW
    for f in range(F):
        pltpu.sync_copy(
            xr_hbm.at[f, pl.ds(base, BPW)], idx_v.at[pl.ds(f * BPW, BPW)]
        )

    flat = table_hbm.at[0]
    pltpu.async_copy(flat.at[idx_v], vals_v, sem).wait()

    def _red(j, _):
        o = j * L
        a = vals_v[pl.ds(o, L)]
        for f in range(1, F):
            a = a + vals_v[pl.ds(f * BPW + o, L)]
        acc_v[pl.ds(o, L)] = a
        return _

    lax.fori_loop(0, OUTV, _red, 0)
    pltpu.sync_copy(acc_v, out_hbm.at[pl.ds(wid * BPW, BPW)])


def kernel(x, table, bias):
    offs = jnp.arange(F, dtype=jnp.int32) * FIELD
    xr = (x + offs[None, :]).T
    out = _fl_kernel(xr, table.reshape(1, TOTAL))
    return out.reshape(B, 1) + bias


# speedup vs baseline: 1.3200x; 1.3200x over previous
import functools

import jax
import jax.numpy as jnp
from jax import lax
from jax.experimental import pallas as pl
from jax.experimental.pallas import tpu as pltpu
from jax.experimental.pallas import tpu_sc as plsc

F = 26
FIELD = 100000
B = 16384
NC, NS, L = 2, 16, 16
NW = NC * NS
BPW = B // NW
E = BPW * F
OUTV = BPW // L
TOTAL = F * FIELD

_mesh = plsc.VectorSubcoreMesh(core_axis_name="c", subcore_axis_name="s")


@functools.partial(
    pl.kernel,
    out_type=jax.ShapeDtypeStruct((B,), jnp.float32),
    mesh=_mesh,
    scratch_types=[
        pltpu.VMEM((E,), jnp.int32),
        pltpu.VMEM((E,), jnp.float32),
        pltpu.VMEM((BPW,), jnp.float32),
        pltpu.SemaphoreType.DMA,
    ],
)
def _fl_kernel(xr_hbm, table_hbm, out_hbm, idx_v, vals_v, acc_v, sem):
    wid = lax.axis_index("s") * NC + lax.axis_index("c")
    base = wid * BPW
    descs = [
        pltpu.async_copy(
            xr_hbm.at[f, pl.ds(base, BPW)], idx_v.at[pl.ds(f * BPW, BPW)], sem
        )
        for f in range(F)
    ]
    for d in descs:
        d.wait()

    flat = table_hbm.at[0]
    pltpu.async_copy(flat.at[idx_v], vals_v, sem).wait()

    def _red(j, _):
        o = j * L
        a = vals_v[pl.ds(o, L)]
        for f in range(1, F):
            a = a + vals_v[pl.ds(f * BPW + o, L)]
        acc_v[pl.ds(o, L)] = a
        return _

    lax.fori_loop(0, OUTV, _red, 0)
    pltpu.sync_copy(acc_v, out_hbm.at[pl.ds(wid * BPW, BPW)])


def kernel(x, table, bias):
    offs = jnp.arange(F, dtype=jnp.int32) * FIELD
    xr = (x + offs[None, :]).T
    out = _fl_kernel(xr, table.reshape(1, TOTAL))
    return out.reshape(B, 1) + bias


# speedup vs baseline: 1.3698x; 1.0377x over previous
import functools

import jax
import jax.numpy as jnp
from jax import lax
from jax.experimental import pallas as pl
from jax.experimental.pallas import tpu as pltpu
from jax.experimental.pallas import tpu_sc as plsc

F = 26
FIELD = 100000
B = 16384
NC, NS, L = 2, 16, 16
NW = NC * NS
BPW = B // NW
E = BPW * F
OUTV = BPW // L
TOTAL = F * FIELD

_mesh = plsc.VectorSubcoreMesh(core_axis_name="c", subcore_axis_name="s")


@functools.partial(
    pl.kernel,
    out_type=jax.ShapeDtypeStruct((1, B), jnp.float32),
    mesh=_mesh,
    scratch_types=[
        pltpu.VMEM((E,), jnp.int32),
        pltpu.VMEM((E,), jnp.float32),
        pltpu.VMEM((BPW,), jnp.float32),
        pltpu.VMEM((L,), jnp.float32),
        pltpu.SemaphoreType.DMA,
        pltpu.SemaphoreType.DMA,
    ],
)
def _fl_kernel(xt_hbm, table_hbm, bias_hbm, out_hbm, idx_v, vals_v, acc_v,
               bias_v, semx, semg):
    wid = lax.axis_index("s") * NC + lax.axis_index("c")
    base = wid * BPW
    flat = table_hbm.at[0]

    pltpu.sync_copy(bias_hbm, bias_v.at[pl.ds(0, 1)])

    xdescs = [
        pltpu.async_copy(
            xt_hbm.at[f, pl.ds(base, BPW)], idx_v.at[pl.ds(f * BPW, BPW)], semx
        )
        for f in range(F)
    ]
    gdescs = []
    for f in range(F):
        xdescs[f].wait()
        gdescs.append(
            pltpu.async_copy(
                flat.at[pl.ds(f * FIELD, FIELD)].at[idx_v.at[pl.ds(f * BPW, BPW)]],
                vals_v.at[pl.ds(f * BPW, BPW)],
                semg,
            )
        )
    for d in gdescs:
        d.wait()

    bvec = bias_v[...].at[jnp.zeros((L,), jnp.int32)].get(
        mode="promise_in_bounds")

    def _red(j, _):
        o = j * L
        a = bvec + vals_v[pl.ds(o, L)]
        for f in range(1, F):
            a = a + vals_v[pl.ds(f * BPW + o, L)]
        acc_v[pl.ds(o, L)] = a
        return _

    lax.fori_loop(0, OUTV, _red, 0)
    pltpu.sync_copy(acc_v, out_hbm.at[0, pl.ds(base, BPW)])


def kernel(x, table, bias):
    out = _fl_kernel(x.T, table.reshape(1, TOTAL), bias)
    return out.reshape(B, 1)
